# baseline (speedup 1.0000x reference)
import functools

import jax
import jax.numpy as jnp
from jax import lax
from jax.experimental import pallas as pl
from jax.experimental.pallas import tpu as pltpu
from jax.experimental.pallas import tpu_sc as plsc

_EPS = 1e-4
_CH = 512
_NBUF = 6
_ACH = 512
_ANBUF = 6

_SC_ROWS = 8192
_NW = 32
_CHR = 32
_SNB = 4


def _reduce_kernel(y_hbm, m_hbm, acc_ref, ybuf, mbuf, ysem, msem):
    rows, C = y_hbm.shape
    nch = (rows - _SC_ROWS) // _CH

    def start(c, slot):
        pltpu.make_async_copy(
            y_hbm.at[pl.ds(c * _CH, _CH), :], ybuf.at[slot], ysem.at[slot]
        ).start()
        pltpu.make_async_copy(
            m_hbm.at[pl.ds(c * _CH, _CH), :], mbuf.at[slot], msem.at[slot]
        ).start()

    for slot in range(_NBUF):
        start(slot, slot)

    dn = (((0,), (0,)), ((), ()))

    def body(c, carry):
        s, sq, n = carry
        slot = jax.lax.rem(c, _NBUF)
        pltpu.make_async_copy(
            y_hbm.at[pl.ds(c * _CH, _CH), :], ybuf.at[slot], ysem.at[slot]
        ).wait()
        pltpu.make_async_copy(
            m_hbm.at[pl.ds(c * _CH, _CH), :], mbuf.at[slot], msem.at[slot]
        ).wait()
        yb = ybuf[slot]
        w = (mbuf[slot] > 0).astype(jnp.float32)
        s = s + jax.lax.dot_general(w, yb, dn, preferred_element_type=jnp.float32)
        sq = sq + jax.lax.dot_general(
            w, yb * yb, dn, preferred_element_type=jnp.float32
        )
        n = n + jnp.sum(w)

        @pl.when(c + _NBUF < nch)
        def _():
            start(c + _NBUF, slot)

        return (s, sq, n)

    z = jnp.zeros((1, C), jnp.float32)
    s, sq, n = jax.lax.fori_loop(0, nch, body, (z, z, jnp.float32(0.0)))
    acc_ref[0:1, :] = s
    acc_ref[1:2, :] = sq
    acc_ref[2:3, :] = jnp.full((1, C), n, jnp.float32)
    acc_ref[3:8, :] = jnp.zeros((5, C), jnp.float32)


def _sc_reduce_body(y_hbm, m_hbm, out_hbm, ybuf, mvec, s_acc, q_acc, osta,
                    sem0, sem1):
    C = 1024
    nvec = C // 16
    rpw = _SC_ROWS // _NW
    wid = lax.axis_index("s") * 2 + lax.axis_index("c")
    base = m_hbm.shape[0] - _SC_ROWS + wid * rpw

    pltpu.sync_copy(m_hbm.at[pl.ds(base, rpw)], mvec.at[pl.ds(0, rpw)])
    zero = jnp.zeros((16,), jnp.float32)
    for j in range(nvec):
        s_acc[pl.ds(j * 16, 16)] = zero
        q_acc[pl.ds(j * 16, 16)] = zero

    sems = (sem0, sem1)
    ngrp = rpw // (_CHR * 2)

    def copy(ch, b):
        return pltpu.make_async_copy(
            y_hbm.at[pl.ds((base + ch * _CHR) * C, _CHR * C)],
            ybuf.at[b],
            sems[b],
        )

    copy(0, 0).start()
    copy(1, 1).start()

    def grp(g, n_vec):
        for b in range(2):
            ch = g * 2 + b
            copy(ch, b).wait()

            def row(rr, nv):
                ridx = ch * _CHR + rr
                msc = mvec[pl.ds(ridx, 16)][0]
                roff = rr * C

                @pl.when(msc > 0)
                def _():
                    for j in range(nvec):
                        x = ybuf[b, pl.ds(roff + j * 16, 16)]
                        plsc.addupdate(s_acc.at[pl.ds(j * 16, 16)], x)
                        plsc.addupdate(q_acc.at[pl.ds(j * 16, 16)], x * x)

                return nv + jnp.where(msc > 0, jnp.float32(1.0),
                                      jnp.float32(0.0))

            n_vec = lax.fori_loop(0, _CHR, row, n_vec)

            @pl.when(ch + 2 < ngrp * 2)
            def _():
                copy(ch + 2, b).start()

        return n_vec

    n_vec = lax.fori_loop(0, ngrp, grp, jnp.zeros((16,), jnp.float32))

    for j in range(nvec):
        osta[pl.ds(j * 16, 16)] = s_acc[pl.ds(j * 16, 16)]
        osta[pl.ds(C + j * 16, 16)] = q_acc[pl.ds(j * 16, 16)]
        osta[pl.ds(2 * C + j * 16, 16)] = n_vec if j == 0 else zero
    pltpu.sync_copy(osta, out_hbm.at[wid])


def _sc_reduce(y2, m1):
    rows, C = y2.shape
    mesh = plsc.VectorSubcoreMesh(core_axis_name="c", subcore_axis_name="s")
    f = pl.kernel(
        _sc_reduce_body,
        mesh=mesh,
        out_type=jax.ShapeDtypeStruct((_NW, 3 * C), jnp.float32),
        scratch_types=[
            pltpu.VMEM((2, _CHR * C), jnp.float32),
            pltpu.VMEM((_SC_ROWS // _NW + 16,), jnp.int32),
            pltpu.VMEM((C,), jnp.float32),
            pltpu.VMEM((C,), jnp.float32),
            pltpu.VMEM((3 * C,), jnp.float32),
            pltpu.SemaphoreType.DMA,
            pltpu.SemaphoreType.DMA,
        ],
    )
    return f(y2.reshape(rows * C), m1)


def _apply_kernel(acc_ref, sc_ref, g_ref, b_ref, y_hbm, m_hbm, o_hbm,
                  ybuf, mbuf, obuf, ysem, msem, osem):
    rows, C = y_hbm.shape
    nch = rows // _ACH
    scp = sc_ref[...]
    s = acc_ref[0, :] + jnp.sum(scp[:, 0:C], axis=0)
    sq = acc_ref[1, :] + jnp.sum(scp[:, C:2 * C], axis=0)
    n = acc_ref[2, 0] + jnp.sum(scp[:, 2 * C])
    mean = s / n
    var = (sq - s * mean) / (n - 1.0)
    std = jnp.sqrt(var)
    scale = g_ref[0, :] / (std + _EPS)
    shift = b_ref[0, :] - mean * scale

    def start_read(c, slot):
        pltpu.make_async_copy(
            y_hbm.at[pl.ds(c * _ACH, _ACH), :], ybuf.at[slot], ysem.at[slot]
        ).start()
        pltpu.make_async_copy(
            m_hbm.at[pl.ds(c * _ACH, _ACH), :], mbuf.at[slot], msem.at[slot]
        ).start()

    for slot in range(_ANBUF):
        start_read(slot, slot)

    def write_copy(c, slot):
        return pltpu.make_async_copy(
            obuf.at[slot], o_hbm.at[pl.ds(c * _ACH, _ACH), :], osem.at[slot]
        )

    def body(c, _):
        slot = jax.lax.rem(c, _ANBUF)
        pltpu.make_async_copy(
            y_hbm.at[pl.ds(c * _ACH, _ACH), :], ybuf.at[slot], ysem.at[slot]
        ).wait()
        pltpu.make_async_copy(
            m_hbm.at[pl.ds(c * _ACH, _ACH), :], mbuf.at[slot], msem.at[slot]
        ).wait()

        @pl.when(c >= _ANBUF)
        def _():
            write_copy(c - _ANBUF, slot).wait()

        yb = ybuf[slot]
        sel = mbuf[slot] > 0
        obuf[slot] = jnp.where(sel, yb * scale + shift, yb)
        write_copy(c, slot).start()

        @pl.when(c + _ANBUF < nch)
        def _():
            start_read(c + _ANBUF, slot)

        return 0

    jax.lax.fori_loop(0, nch, body, 0)
    for slot in range(_ANBUF):
        c = nch - _ANBUF + slot
        pltpu.make_async_copy(
            obuf.at[jax.lax.rem(c, _ANBUF)],
            o_hbm.at[pl.ds(c * _ACH, _ACH), :],
            osem.at[jax.lax.rem(c, _ANBUF)],
        ).wait()


def kernel(y, mask, gamma, beta):
    B, T, C = y.shape
    rows = B * T
    y2 = y.reshape(rows, C)
    m2 = mask.reshape(rows, 1)
    m1 = mask.reshape(rows)

    sc_acc = _sc_reduce(y2, m1)

    acc = pl.pallas_call(
        _reduce_kernel,
        in_specs=[
            pl.BlockSpec(memory_space=pl.ANY),
            pl.BlockSpec(memory_space=pl.ANY),
        ],
        out_specs=pl.BlockSpec(memory_space=pltpu.VMEM),
        out_shape=jax.ShapeDtypeStruct((8, C), jnp.float32),
        scratch_shapes=[
            pltpu.VMEM((_NBUF, _CH, C), jnp.float32),
            pltpu.VMEM((_NBUF, _CH, 1), jnp.int32),
            pltpu.SemaphoreType.DMA((_NBUF,)),
            pltpu.SemaphoreType.DMA((_NBUF,)),
        ],
        compiler_params=pltpu.CompilerParams(
            vmem_limit_bytes=120 * 1024 * 1024,
        ),
    )(y2, m2)

    out = pl.pallas_call(
        _apply_kernel,
        in_specs=[
            pl.BlockSpec(memory_space=pltpu.VMEM),
            pl.BlockSpec(memory_space=pltpu.VMEM),
            pl.BlockSpec(memory_space=pltpu.VMEM),
            pl.BlockSpec(memory_space=pltpu.VMEM),
            pl.BlockSpec(memory_space=pl.ANY),
            pl.BlockSpec(memory_space=pl.ANY),
        ],
        out_specs=pl.BlockSpec(memory_space=pl.ANY),
        out_shape=jax.ShapeDtypeStruct((rows, C), jnp.float32),
        scratch_shapes=[
            pltpu.VMEM((_ANBUF, _ACH, C), jnp.float32),
            pltpu.VMEM((_ANBUF, _ACH, 1), jnp.int32),
            pltpu.VMEM((_ANBUF, _ACH, C), jnp.float32),
            pltpu.SemaphoreType.DMA((_ANBUF,)),
            pltpu.SemaphoreType.DMA((_ANBUF,)),
            pltpu.SemaphoreType.DMA((_ANBUF,)),
        ],
        compiler_params=pltpu.CompilerParams(
            vmem_limit_bytes=120 * 1024 * 1024,
        ),
    )(acc, sc_acc, gamma.reshape(1, C), beta.reshape(1, C), y2, m2)

    return out.reshape(B, T, C)


# speedup vs baseline: 1.6783x; 1.6783x over previous
import functools

import jax
import jax.numpy as jnp
from jax import lax
from jax.experimental import pallas as pl
from jax.experimental.pallas import tpu as pltpu
from jax.experimental.pallas import tpu_sc as plsc

_EPS = 1e-4
_CH = 512
_NBUF = 6
_ACH = 512
_ANBUF = 6

_SC_ROWS = 3072
_NW = 32
_CHR = 16
_SNB = 4


def _reduce_kernel(y_hbm, m_hbm, acc_ref, ybuf, mbuf, ysem, msem):
    rows, C = y_hbm.shape
    nch = (rows - _SC_ROWS) // _CH

    def start(c, slot):
        pltpu.make_async_copy(
            y_hbm.at[pl.ds(c * _CH, _CH), :], ybuf.at[slot], ysem.at[slot]
        ).start()
        pltpu.make_async_copy(
            m_hbm.at[pl.ds(c * _CH, _CH), :], mbuf.at[slot], msem.at[slot]
        ).start()

    for slot in range(_NBUF):
        start(slot, slot)

    dn = (((0,), (0,)), ((), ()))

    def body(c, carry):
        s, sq, n = carry
        slot = jax.lax.rem(c, _NBUF)
        pltpu.make_async_copy(
            y_hbm.at[pl.ds(c * _CH, _CH), :], ybuf.at[slot], ysem.at[slot]
        ).wait()
        pltpu.make_async_copy(
            m_hbm.at[pl.ds(c * _CH, _CH), :], mbuf.at[slot], msem.at[slot]
        ).wait()
        yb = ybuf[slot]
        w = (mbuf[slot] > 0).astype(jnp.float32)
        s = s + jax.lax.dot_general(w, yb, dn, preferred_element_type=jnp.float32)
        sq = sq + jax.lax.dot_general(
            w, yb * yb, dn, preferred_element_type=jnp.float32
        )
        n = n + jnp.sum(w)

        @pl.when(c + _NBUF < nch)
        def _():
            start(c + _NBUF, slot)

        return (s, sq, n)

    z = jnp.zeros((1, C), jnp.float32)
    s, sq, n = jax.lax.fori_loop(0, nch, body, (z, z, jnp.float32(0.0)))
    acc_ref[0:1, :] = s
    acc_ref[1:2, :] = sq
    acc_ref[2:3, :] = jnp.full((1, C), n, jnp.float32)
    acc_ref[3:8, :] = jnp.zeros((5, C), jnp.float32)


def _sc_reduce_body(y_hbm, m_hbm, out_hbm, ybuf, mvec, s_acc, q_acc, osta,
                    sem0, sem1):
    C = y_hbm.shape[1]
    nvec = C // 16
    rpw = _SC_ROWS // _NW
    wid = lax.axis_index("s") * 2 + lax.axis_index("c")
    base = y_hbm.shape[0] - _SC_ROWS + wid * rpw

    pltpu.sync_copy(m_hbm.at[pl.ds(base, rpw)], mvec.at[pl.ds(0, rpw)])
    zero = jnp.zeros((16,), jnp.float32)
    for j in range(nvec):
        s_acc[pl.ds(j * 16, 16)] = zero
        q_acc[pl.ds(j * 16, 16)] = zero

    sems = (sem0, sem1)
    ngrp = rpw // (_CHR * 2)

    def copy(ch, b):
        return pltpu.make_async_copy(
            y_hbm.at[pl.ds(base + ch * _CHR, _CHR), :], ybuf.at[b], sems[b]
        )

    copy(0, 0).start()
    copy(1, 1).start()

    def grp(g, n_vec):
        for b in range(2):
            ch = g * 2 + b
            copy(ch, b).wait()

            def row(rr, nv):
                ridx = ch * _CHR + rr
                msc = mvec[pl.ds(ridx, 16)][0]
                ws = jnp.where(msc > 0, jnp.float32(1.0), jnp.float32(0.0))
                for j in range(nvec):
                    x = ybuf[b, rr, pl.ds(j * 16, 16)]
                    xs = x * ws
                    plsc.addupdate(s_acc.at[pl.ds(j * 16, 16)], xs)
                    plsc.addupdate(q_acc.at[pl.ds(j * 16, 16)], xs * x)
                return nv + ws

            n_vec = lax.fori_loop(0, _CHR, row, n_vec)

            @pl.when(ch + 2 < ngrp * 2)
            def _():
                copy(ch + 2, b).start()

        return n_vec

    n_vec = lax.fori_loop(0, ngrp, grp, jnp.zeros((16,), jnp.float32))

    for j in range(nvec):
        osta[pl.ds(j * 16, 16)] = s_acc[pl.ds(j * 16, 16)]
        osta[pl.ds(C + j * 16, 16)] = q_acc[pl.ds(j * 16, 16)]
        osta[pl.ds(2 * C + j * 16, 16)] = n_vec if j == 0 else zero
    pltpu.sync_copy(osta, out_hbm.at[wid])


def _sc_reduce(y2, m1):
    rows, C = y2.shape
    mesh = plsc.VectorSubcoreMesh(core_axis_name="c", subcore_axis_name="s")
    f = pl.kernel(
        _sc_reduce_body,
        mesh=mesh,
        out_type=jax.ShapeDtypeStruct((_NW, 3 * C), jnp.float32),
        scratch_types=[
            pltpu.VMEM((2, _CHR, C), jnp.float32),
            pltpu.VMEM((_SC_ROWS // _NW + 16,), jnp.int32),
            pltpu.VMEM((C,), jnp.float32),
            pltpu.VMEM((C,), jnp.float32),
            pltpu.VMEM((3 * C,), jnp.float32),
            pltpu.SemaphoreType.DMA,
            pltpu.SemaphoreType.DMA,
        ],
    )
    return f(y2, m1)


def _apply_kernel(acc_ref, sc_ref, g_ref, b_ref, y_hbm, m_hbm, o_hbm,
                  ybuf, mbuf, obuf, ysem, msem, osem):
    rows, C = y_hbm.shape
    nch = rows // _ACH
    scp = sc_ref[...]
    s = acc_ref[0, :] + jnp.sum(scp[:, 0:C], axis=0)
    sq = acc_ref[1, :] + jnp.sum(scp[:, C:2 * C], axis=0)
    n = acc_ref[2, 0] + jnp.sum(scp[:, 2 * C])
    mean = s / n
    var = (sq - s * mean) / (n - 1.0)
    std = jnp.sqrt(var)
    scale = g_ref[0, :] / (std + _EPS)
    shift = b_ref[0, :] - mean * scale

    def start_read(c, slot):
        pltpu.make_async_copy(
            y_hbm.at[pl.ds(c * _ACH, _ACH), :], ybuf.at[slot], ysem.at[slot]
        ).start()
        pltpu.make_async_copy(
            m_hbm.at[pl.ds(c * _ACH, _ACH), :], mbuf.at[slot], msem.at[slot]
        ).start()

    for slot in range(_ANBUF):
        start_read(slot, slot)

    def write_copy(c, slot):
        return pltpu.make_async_copy(
            obuf.at[slot], o_hbm.at[pl.ds(c * _ACH, _ACH), :], osem.at[slot]
        )

    def body(c, _):
        slot = jax.lax.rem(c, _ANBUF)
        pltpu.make_async_copy(
            y_hbm.at[pl.ds(c * _ACH, _ACH), :], ybuf.at[slot], ysem.at[slot]
        ).wait()
        pltpu.make_async_copy(
            m_hbm.at[pl.ds(c * _ACH, _ACH), :], mbuf.at[slot], msem.at[slot]
        ).wait()

        @pl.when(c >= _ANBUF)
        def _():
            write_copy(c - _ANBUF, slot).wait()

        yb = ybuf[slot]
        sel = mbuf[slot] > 0
        obuf[slot] = jnp.where(sel, yb * scale + shift, yb)
        write_copy(c, slot).start()

        @pl.when(c + _ANBUF < nch)
        def _():
            start_read(c + _ANBUF, slot)

        return 0

    jax.lax.fori_loop(0, nch, body, 0)
    for slot in range(_ANBUF):
        c = nch - _ANBUF + slot
        pltpu.make_async_copy(
            obuf.at[jax.lax.rem(c, _ANBUF)],
            o_hbm.at[pl.ds(c * _ACH, _ACH), :],
            osem.at[jax.lax.rem(c, _ANBUF)],
        ).wait()


def kernel(y, mask, gamma, beta):
    B, T, C = y.shape
    rows = B * T
    y2 = y.reshape(rows, C)
    m2 = mask.reshape(rows, 1)
    m1 = mask.reshape(rows)

    sc_acc = _sc_reduce(y2, m1)

    acc = pl.pallas_call(
        _reduce_kernel,
        in_specs=[
            pl.BlockSpec(memory_space=pl.ANY),
            pl.BlockSpec(memory_space=pl.ANY),
        ],
        out_specs=pl.BlockSpec(memory_space=pltpu.VMEM),
        out_shape=jax.ShapeDtypeStruct((8, C), jnp.float32),
        scratch_shapes=[
            pltpu.VMEM((_NBUF, _CH, C), jnp.float32),
            pltpu.VMEM((_NBUF, _CH, 1), jnp.int32),
            pltpu.SemaphoreType.DMA((_NBUF,)),
            pltpu.SemaphoreType.DMA((_NBUF,)),
        ],
        compiler_params=pltpu.CompilerParams(
            vmem_limit_bytes=120 * 1024 * 1024,
        ),
    )(y2, m2)

    out = pl.pallas_call(
        _apply_kernel,
        in_specs=[
            pl.BlockSpec(memory_space=pltpu.VMEM),
            pl.BlockSpec(memory_space=pltpu.VMEM),
            pl.BlockSpec(memory_space=pltpu.VMEM),
            pl.BlockSpec(memory_space=pltpu.VMEM),
            pl.BlockSpec(memory_space=pl.ANY),
            pl.BlockSpec(memory_space=pl.ANY),
        ],
        out_specs=pl.BlockSpec(memory_space=pl.ANY),
        out_shape=jax.ShapeDtypeStruct((rows, C), jnp.float32),
        scratch_shapes=[
            pltpu.VMEM((_ANBUF, _ACH, C), jnp.float32),
            pltpu.VMEM((_ANBUF, _ACH, 1), jnp.int32),
            pltpu.VMEM((_ANBUF, _ACH, C), jnp.float32),
            pltpu.SemaphoreType.DMA((_ANBUF,)),
            pltpu.SemaphoreType.DMA((_ANBUF,)),
            pltpu.SemaphoreType.DMA((_ANBUF,)),
        ],
        compiler_params=pltpu.CompilerParams(
            vmem_limit_bytes=120 * 1024 * 1024,
        ),
    )(acc, sc_acc, gamma.reshape(1, C), beta.reshape(1, C), y2, m2)

    return out.reshape(B, T, C)
